# baseline (speedup 1.0000x reference)
import functools
import jax
import jax.numpy as jnp
from jax import lax
from jax.experimental import pallas as pl
from jax.experimental.pallas import tpu as pltpu
from jax.experimental.pallas import tpu_sc as plsc

BS, C, T, N = 32, 16, 24, 128
R = BS * C
E = N * N
NC, NS, L = 2, 16, 16
NW = NC * NS
EPW = E // NW
RPW = R // NW
CHUNK = 1024
NI = CHUNK // N
NCHUNK = E // CHUNK

_mesh = plsc.VectorSubcoreMesh(
    core_axis_name="c", subcore_axis_name="s", num_cores=NC, num_subcores=NS)


def _tc_body(xs_ref, yl_ref, wt_ref, par_ref, src_ref, dst_ref,
             p_ref, q_ref):
    acc = jnp.zeros((R, N), jnp.float32)
    for t in range(T - 1):
        acc = acc + xs_ref[:, t, :] * wt_ref[:, t].reshape(R, 1)

    def p(i):
        return par_ref[i, :].reshape(R, 1)

    tcm_b, w0, w1, be = p(0), p(1), p(2), p(3)
    e2s_w, e2s_b, e2d_w, e2d_b = p(4), p(5), p(6), p(7)
    w20, w21, w22, b2 = p(8), p(9), p(10), p(11)

    A = (acc + tcm_b) * w0
    B = yl_ref[...] * w1

    iota_n = lax.broadcasted_iota(jnp.int32, (E, N), 1)
    S1 = (src_ref[...] == iota_n).astype(jnp.bfloat16)
    D1 = (dst_ref[...] == iota_n).astype(jnp.bfloat16)
    M = lax.dot_general(S1, D1, (((0,), (0,)), ((), ())),
                        preferred_element_type=jnp.float32)
    cs = jnp.sum(M, axis=1).reshape(1, N)
    cd = jnp.sum(M, axis=0).reshape(1, N)
    SB = lax.dot_general(B, M, (((1,), (1,)), ((), ())),
                         preferred_element_type=jnp.float32)
    SA = lax.dot_general(A, M, (((1,), (0,)), ((), ())),
                         preferred_element_type=jnp.float32)

    inv_cs = 1.0 / jnp.maximum(cs, 1.0)
    inv_cd = 1.0 / jnp.maximum(cd, 1.0)
    ms = (cs * (A + be) + SB) * inv_cs
    md = (cd * (B + be) + SA) * inv_cd

    def lk(v):
        return jnp.maximum(v, 0.01 * v)

    xc = lk(ms * e2s_w + e2s_b)
    yr = lk(md * e2d_w + e2d_b)

    P = w20 * xc + w22 * A + (w22 * be + b2)
    Q = w21 * yr + w22 * B

    def pack_pairs(V):
        v3 = V.reshape(R // 2, 2, N)
        lo = lax.bitcast_convert_type(
            v3[:, 0, :].astype(jnp.bfloat16), jnp.uint16).astype(jnp.uint32)
        hi = lax.bitcast_convert_type(
            v3[:, 1, :].astype(jnp.bfloat16), jnp.uint16).astype(jnp.uint32)
        return lax.bitcast_convert_type(lo | (hi << 16), jnp.int32)

    p_ref[...] = pack_pairs(P)
    q_ref[...] = pack_pairs(Q)


_tc_dense = pl.pallas_call(
    _tc_body,
    out_shape=(jax.ShapeDtypeStruct((R // 2, N), jnp.int32),
               jax.ShapeDtypeStruct((R // 2, N), jnp.int32)),
)

@functools.partial(
    pl.kernel,
    out_type=jax.ShapeDtypeStruct((BS, C, N, N), jnp.float32),
    mesh=_mesh,
    compiler_params=pltpu.CompilerParams(needs_layout_passes=False),
    scratch_types=[
        pltpu.VMEM((E,), jnp.int32),
        pltpu.VMEM((E,), jnp.int32),
        pltpu.VMEM((RPW // 2 * N,), jnp.int32),
        pltpu.VMEM((RPW // 2 * N,), jnp.int32),
        pltpu.VMEM((RPW, NI, N), jnp.float32),
        pltpu.VMEM((RPW, NI, N), jnp.float32),
        pltpu.SemaphoreType.DMA,
        pltpu.SemaphoreType.DMA,
    ],
)
def _sc_gather(p_hbm, q_hbm, src_hbm, dst_hbm, out_hbm,
               src_v, dst_v, p_v, q_v, ob0, ob1, sem0, sem1):
    wid = lax.axis_index("s") * NC + lax.axis_index("c")
    npw = RPW // 2 * N
    pltpu.async_copy(src_hbm, src_v, sem0)
    pltpu.async_copy(dst_hbm, dst_v, sem1)
    pltpu.async_copy(p_hbm.at[pl.ds(wid * npw, npw)], p_v, sem0)
    pltpu.async_copy(q_hbm.at[pl.ds(wid * npw, npw)], q_v, sem1)
    pltpu.make_async_copy(src_hbm, src_v, sem0).wait()
    pltpu.make_async_copy(dst_hbm, dst_v, sem1).wait()
    pltpu.make_async_copy(p_hbm.at[pl.ds(wid * npw, npw)], p_v, sem0).wait()
    pltpu.make_async_copy(q_hbm.at[pl.ds(wid * npw, npw)], q_v, sem1).wait()

    gpc = CHUNK // L

    def fill(ob, ci):
        @plsc.parallel_loop(0, gpc, 1, unroll=4)
        def _(it):
            si = it >> 3
            v2 = it & 7
            off = ci * CHUNK + it * L
            sv = src_v[pl.ds(off, L)]
            dv = dst_v[pl.ds(off, L)]
            for k in range(RPW // 2):
                pw = plsc.load_gather(p_v, [sv + (k * N)])
                qw = plsc.load_gather(q_v, [dv + (k * N)])
                pa, pb = plsc.unpack(
                    plsc.bitcast(pw, jnp.bfloat16),
                    format=plsc.PackFormat.INTERLEAVED,
                    preferred_element_type=jnp.float32)
                qa, qb = plsc.unpack(
                    plsc.bitcast(qw, jnp.bfloat16),
                    format=plsc.PackFormat.INTERLEAVED,
                    preferred_element_type=jnp.float32)
                s0 = pa + qa
                s1 = pb + qb
                ob[2 * k, si, pl.ds(v2 * L, L)] = jnp.maximum(s0, 0.01 * s0)
                ob[2 * k + 1, si, pl.ds(v2 * L, L)] = jnp.maximum(s1, 0.01 * s1)

    def osl(ci):
        return out_hbm.at[wid, :, pl.ds(ci * NI, NI), :]

    def k_body(k, _):
        ci0, ci1 = 2 * k, 2 * k + 1

        @pl.when(k > 0)
        def _():
            pltpu.make_async_copy(ob0, osl(ci0), sem0).wait()

        fill(ob0, ci0)
        pltpu.async_copy(ob0, osl(ci0), sem0)

        @pl.when(k > 0)
        def _():
            pltpu.make_async_copy(ob1, osl(ci1), sem1).wait()

        fill(ob1, ci1)
        pltpu.async_copy(ob1, osl(ci1), sem1)
        return 0

    lax.fori_loop(0, NCHUNK // 2, k_body, 0)
    pltpu.make_async_copy(ob0, osl(0), sem0).wait()
    pltpu.make_async_copy(ob1, osl(1), sem1).wait()


def kernel(x, y, src, dst, tcm_w, tcm_b, n2e_w, n2e_b,
           e2s_w, e2s_b, e2d_w, e2d_b, n2e2_w, n2e2_b):
    xs = x.reshape(R, T, N)
    y_last = y[:, :, T - 1, :].reshape(R, N)
    wt_r = jnp.tile(tcm_w.reshape(1, C, T - 1), (BS, 1, 1)).reshape(R, T - 1)

    def per_row(v):
        return jnp.tile(v.reshape(1, C), (BS, 1)).reshape(R)

    params = jnp.stack([
        per_row(tcm_b),
        per_row(n2e_w[:, 0, 0, 0]), per_row(n2e_w[:, 0, 0, 1]),
        per_row(n2e_b),
        per_row(e2s_w[:, 0, 0, 0]), per_row(e2s_b),
        per_row(e2d_w[:, 0, 0, 0]), per_row(e2d_b),
        per_row(n2e2_w[:, 0, 0, 0]), per_row(n2e2_w[:, 0, 0, 1]),
        per_row(n2e2_w[:, 0, 0, 2]), per_row(n2e2_b),
    ])

    Pp, Qp = _tc_dense(xs, y_last, wt_r, params,
                       src.reshape(E, 1), dst.reshape(E, 1))
    return _sc_gather(Pp.reshape(R // 2 * N), Qp.reshape(R // 2 * N),
                      src, dst)


# speedup vs baseline: 1.2421x; 1.2421x over previous
import functools
import jax
import jax.numpy as jnp
from jax import lax
from jax.experimental import pallas as pl
from jax.experimental.pallas import tpu as pltpu
from jax.experimental.pallas import tpu_sc as plsc

BS, C, T, N = 32, 16, 24, 128
R = BS * C
E = N * N
NC, NS, L = 2, 16, 16
NW = NC * NS
EPW = E // NW
RPW = R // NW
CHUNK = 1024
NI = CHUNK // N
NCHUNK = E // CHUNK

_mesh = plsc.VectorSubcoreMesh(
    core_axis_name="c", subcore_axis_name="s", num_cores=NC, num_subcores=NS)


@functools.partial(
    pl.kernel,
    out_type=jax.ShapeDtypeStruct((NW, E), jnp.float32),
    mesh=_mesh,
    compiler_params=pltpu.CompilerParams(needs_layout_passes=False),
    scratch_types=[
        pltpu.VMEM((EPW,), jnp.int32),
        pltpu.VMEM((EPW,), jnp.int32),
        pltpu.VMEM((E,), jnp.float32),
    ],
)
def _sc_hist(src_hbm, dst_hbm, out_hbm, src_v, dst_v, hist_v):
    wid = lax.axis_index("s") * NC + lax.axis_index("c")
    base = wid * EPW
    pltpu.sync_copy(src_hbm.at[pl.ds(base, EPW)], src_v)
    pltpu.sync_copy(dst_hbm.at[pl.ds(base, EPW)], dst_v)

    zeros = jnp.zeros((L,), jnp.float32)

    def zero_body(i, _):
        for u in range(8):
            hist_v[pl.ds(i * (8 * L) + u * L, L)] = zeros
        return 0

    lax.fori_loop(0, E // (8 * L), zero_body, 0)

    ones = jnp.ones((L,), jnp.float32)

    def scat_body(j, _):
        sv = src_v[pl.ds(j * L, L)]
        dv = dst_v[pl.ds(j * L, L)]
        plsc.addupdate_scatter(hist_v, [sv * N + dv], ones)
        return 0

    lax.fori_loop(0, EPW // L, scat_body, 0)
    pltpu.sync_copy(hist_v, out_hbm.at[wid])


def _tc_body(xs_ref, yl_ref, wt_ref, par_ref, hp_ref, p_ref, q_ref):
    acc = jnp.zeros((R, N), jnp.float32)
    for t in range(T - 1):
        acc = acc + xs_ref[:, t, :] * wt_ref[:, t].reshape(R, 1)

    def p(i):
        return par_ref[i, :].reshape(R, 1)

    tcm_b, w0, w1, be = p(0), p(1), p(2), p(3)
    e2s_w, e2s_b, e2d_w, e2d_b = p(4), p(5), p(6), p(7)
    w20, w21, w22, b2 = p(8), p(9), p(10), p(11)

    A = (acc + tcm_b) * w0
    B = yl_ref[...] * w1

    M = jnp.sum(hp_ref[...], axis=0)
    cs = jnp.sum(M, axis=1).reshape(1, N)
    cd = jnp.sum(M, axis=0).reshape(1, N)
    SB = lax.dot_general(B, M, (((1,), (1,)), ((), ())),
                         preferred_element_type=jnp.float32)
    SA = lax.dot_general(A, M, (((1,), (0,)), ((), ())),
                         preferred_element_type=jnp.float32)

    inv_cs = 1.0 / jnp.maximum(cs, 1.0)
    inv_cd = 1.0 / jnp.maximum(cd, 1.0)
    ms = (cs * (A + be) + SB) * inv_cs
    md = (cd * (B + be) + SA) * inv_cd

    def lk(v):
        return jnp.maximum(v, 0.01 * v)

    xc = lk(ms * e2s_w + e2s_b)
    yr = lk(md * e2d_w + e2d_b)

    P = w20 * xc + w22 * A + (w22 * be + b2)
    Q = w21 * yr + w22 * B

    def pack_pairs(V):
        v3 = V.reshape(R // 2, 2, N)
        lo = lax.bitcast_convert_type(
            v3[:, 0, :].astype(jnp.bfloat16), jnp.uint16).astype(jnp.uint32)
        hi = lax.bitcast_convert_type(
            v3[:, 1, :].astype(jnp.bfloat16), jnp.uint16).astype(jnp.uint32)
        return lax.bitcast_convert_type(lo | (hi << 16), jnp.int32)

    p_ref[...] = pack_pairs(P)
    q_ref[...] = pack_pairs(Q)


_tc_dense = pl.pallas_call(
    _tc_body,
    out_shape=(jax.ShapeDtypeStruct((R // 2, N), jnp.int32),
               jax.ShapeDtypeStruct((R // 2, N), jnp.int32)),
)

@functools.partial(
    pl.kernel,
    out_type=jax.ShapeDtypeStruct((BS, C, N, N), jnp.float32),
    mesh=_mesh,
    compiler_params=pltpu.CompilerParams(needs_layout_passes=False),
    scratch_types=[
        pltpu.VMEM((E,), jnp.int32),
        pltpu.VMEM((E,), jnp.int32),
        pltpu.VMEM((RPW // 2 * N,), jnp.int32),
        pltpu.VMEM((RPW // 2 * N,), jnp.int32),
        pltpu.VMEM((RPW, NI, N), jnp.float32),
        pltpu.VMEM((RPW, NI, N), jnp.float32),
        pltpu.SemaphoreType.DMA,
        pltpu.SemaphoreType.DMA,
    ],
)
def _sc_gather(p_hbm, q_hbm, src_hbm, dst_hbm, out_hbm,
               src_v, dst_v, p_v, q_v, ob0, ob1, sem0, sem1):
    wid = lax.axis_index("s") * NC + lax.axis_index("c")
    npw = RPW // 2 * N
    pltpu.async_copy(src_hbm, src_v, sem0)
    pltpu.async_copy(dst_hbm, dst_v, sem1)
    pltpu.async_copy(p_hbm.at[pl.ds(wid * npw, npw)], p_v, sem0)
    pltpu.async_copy(q_hbm.at[pl.ds(wid * npw, npw)], q_v, sem1)
    pltpu.make_async_copy(src_hbm, src_v, sem0).wait()
    pltpu.make_async_copy(dst_hbm, dst_v, sem1).wait()
    pltpu.make_async_copy(p_hbm.at[pl.ds(wid * npw, npw)], p_v, sem0).wait()
    pltpu.make_async_copy(q_hbm.at[pl.ds(wid * npw, npw)], q_v, sem1).wait()

    gpc = CHUNK // L

    def fill(ob, ci):
        @plsc.parallel_loop(0, gpc, 1, unroll=4)
        def _(it):
            si = it >> 3
            v2 = it & 7
            off = ci * CHUNK + it * L
            sv = src_v[pl.ds(off, L)]
            dv = dst_v[pl.ds(off, L)]
            for k in range(RPW // 2):
                pw = plsc.load_gather(p_v, [sv + (k * N)])
                qw = plsc.load_gather(q_v, [dv + (k * N)])
                pa, pb = plsc.unpack(
                    plsc.bitcast(pw, jnp.bfloat16),
                    format=plsc.PackFormat.INTERLEAVED,
                    preferred_element_type=jnp.float32)
                qa, qb = plsc.unpack(
                    plsc.bitcast(qw, jnp.bfloat16),
                    format=plsc.PackFormat.INTERLEAVED,
                    preferred_element_type=jnp.float32)
                s0 = pa + qa
                s1 = pb + qb
                ob[2 * k, si, pl.ds(v2 * L, L)] = jnp.maximum(s0, 0.01 * s0)
                ob[2 * k + 1, si, pl.ds(v2 * L, L)] = jnp.maximum(s1, 0.01 * s1)

    def osl(ci):
        return out_hbm.at[wid, :, pl.ds(ci * NI, NI), :]

    def k_body(k, _):
        ci0, ci1 = 2 * k, 2 * k + 1

        @pl.when(k > 0)
        def _():
            pltpu.make_async_copy(ob0, osl(ci0), sem0).wait()

        fill(ob0, ci0)
        pltpu.async_copy(ob0, osl(ci0), sem0)

        @pl.when(k > 0)
        def _():
            pltpu.make_async_copy(ob1, osl(ci1), sem1).wait()

        fill(ob1, ci1)
        pltpu.async_copy(ob1, osl(ci1), sem1)
        return 0

    lax.fori_loop(0, NCHUNK // 2, k_body, 0)
    pltpu.make_async_copy(ob0, osl(0), sem0).wait()
    pltpu.make_async_copy(ob1, osl(1), sem1).wait()


def kernel(x, y, src, dst, tcm_w, tcm_b, n2e_w, n2e_b,
           e2s_w, e2s_b, e2d_w, e2d_b, n2e2_w, n2e2_b):
    xs = x.reshape(R, T, N)
    y_last = y[:, :, T - 1, :].reshape(R, N)
    wt_r = jnp.tile(tcm_w.reshape(1, C, T - 1), (BS, 1, 1)).reshape(R, T - 1)

    def per_row(v):
        return jnp.tile(v.reshape(1, C), (BS, 1)).reshape(R)

    params = jnp.stack([
        per_row(tcm_b),
        per_row(n2e_w[:, 0, 0, 0]), per_row(n2e_w[:, 0, 0, 1]),
        per_row(n2e_b),
        per_row(e2s_w[:, 0, 0, 0]), per_row(e2s_b),
        per_row(e2d_w[:, 0, 0, 0]), per_row(e2d_b),
        per_row(n2e2_w[:, 0, 0, 0]), per_row(n2e2_w[:, 0, 0, 1]),
        per_row(n2e2_w[:, 0, 0, 2]), per_row(n2e2_b),
    ])

    partials = _sc_hist(src, dst)
    Pp, Qp = _tc_dense(xs, y_last, wt_r, params,
                       partials.reshape(NW, N, N))
    return _sc_gather(Pp.reshape(R // 2 * N), Qp.reshape(R // 2 * N),
                      src, dst)


# speedup vs baseline: 1.2449x; 1.0022x over previous
import functools
import jax
import jax.numpy as jnp
from jax import lax
from jax.experimental import pallas as pl
from jax.experimental.pallas import tpu as pltpu
from jax.experimental.pallas import tpu_sc as plsc

BS, C, T, N = 32, 16, 24, 128
R = BS * C
E = N * N
NC, NS, L = 2, 16, 16
NW = NC * NS
EPW = E // NW
RPW = R // NW
CHUNK = 1024
NI = CHUNK // N
NCHUNK = E // CHUNK

_mesh = plsc.VectorSubcoreMesh(
    core_axis_name="c", subcore_axis_name="s", num_cores=NC, num_subcores=NS)


@functools.partial(
    pl.kernel,
    out_type=jax.ShapeDtypeStruct((NW, E), jnp.float32),
    mesh=_mesh,
    compiler_params=pltpu.CompilerParams(needs_layout_passes=False),
    scratch_types=[
        pltpu.VMEM((EPW,), jnp.int32),
        pltpu.VMEM((EPW,), jnp.int32),
        pltpu.VMEM((E,), jnp.float32),
    ],
)
def _sc_hist(src_hbm, dst_hbm, out_hbm, src_v, dst_v, hist_v):
    wid = lax.axis_index("s") * NC + lax.axis_index("c")
    base = wid * EPW
    pltpu.sync_copy(src_hbm.at[pl.ds(base, EPW)], src_v)
    pltpu.sync_copy(dst_hbm.at[pl.ds(base, EPW)], dst_v)

    zeros = jnp.zeros((L,), jnp.float32)

    def zero_body(i, _):
        for u in range(8):
            hist_v[pl.ds(i * (8 * L) + u * L, L)] = zeros
        return 0

    lax.fori_loop(0, E // (8 * L), zero_body, 0)

    ones = jnp.ones((L,), jnp.float32)

    def scat_body(j, _):
        sv = src_v[pl.ds(j * L, L)]
        dv = dst_v[pl.ds(j * L, L)]
        plsc.addupdate_scatter(hist_v, [sv * N + dv], ones)
        return 0

    lax.fori_loop(0, EPW // L, scat_body, 0)
    pltpu.sync_copy(hist_v, out_hbm.at[wid])


def _tc_body(xs_ref, yl_ref, wt_ref, par_ref, hp_ref, p_ref, q_ref):
    acc = jnp.zeros((R, N), jnp.float32)
    for t in range(T - 1):
        acc = acc + xs_ref[:, t, :] * wt_ref[:, t].reshape(R, 1)

    def p(i):
        return par_ref[:, i:i + 1]

    tcm_b, w0, w1, be = p(0), p(1), p(2), p(3)
    e2s_w, e2s_b, e2d_w, e2d_b = p(4), p(5), p(6), p(7)
    w20, w21, w22, b2 = p(8), p(9), p(10), p(11)

    A = (acc + tcm_b) * w0
    B = yl_ref[...] * w1

    M = jnp.sum(hp_ref[...], axis=0)
    cs = jnp.sum(M, axis=1).reshape(1, N)
    cd = jnp.sum(M, axis=0).reshape(1, N)
    SB = lax.dot_general(B, M, (((1,), (1,)), ((), ())),
                         preferred_element_type=jnp.float32)
    SA = lax.dot_general(A, M, (((1,), (0,)), ((), ())),
                         preferred_element_type=jnp.float32)

    inv_cs = 1.0 / jnp.maximum(cs, 1.0)
    inv_cd = 1.0 / jnp.maximum(cd, 1.0)
    ms = (cs * (A + be) + SB) * inv_cs
    md = (cd * (B + be) + SA) * inv_cd

    def lk(v):
        return jnp.maximum(v, 0.01 * v)

    xc = lk(ms * e2s_w + e2s_b)
    yr = lk(md * e2d_w + e2d_b)

    P = w20 * xc + w22 * A + (w22 * be + b2)
    Q = w21 * yr + w22 * B

    def pack_pairs(V):
        v4 = V.reshape(NW, 2, RPW // 2, N)
        lo = lax.bitcast_convert_type(
            v4[:, 0].reshape(R // 2, N).astype(jnp.bfloat16),
            jnp.uint16).astype(jnp.uint32)
        hi = lax.bitcast_convert_type(
            v4[:, 1].reshape(R // 2, N).astype(jnp.bfloat16),
            jnp.uint16).astype(jnp.uint32)
        return lax.bitcast_convert_type(lo | (hi << 16), jnp.int32)

    p_ref[...] = pack_pairs(P)
    q_ref[...] = pack_pairs(Q)


_tc_dense = pl.pallas_call(
    _tc_body,
    out_shape=(jax.ShapeDtypeStruct((R // 2, N), jnp.int32),
               jax.ShapeDtypeStruct((R // 2, N), jnp.int32)),
)

@functools.partial(
    pl.kernel,
    out_type=jax.ShapeDtypeStruct((BS, C, N, N), jnp.float32),
    mesh=_mesh,
    compiler_params=pltpu.CompilerParams(needs_layout_passes=False),
    scratch_types=[
        pltpu.VMEM((E,), jnp.int32),
        pltpu.VMEM((E,), jnp.int32),
        pltpu.VMEM((RPW // 2 * N,), jnp.int32),
        pltpu.VMEM((RPW // 2 * N,), jnp.int32),
        pltpu.VMEM((RPW, NI, N), jnp.float32),
        pltpu.VMEM((RPW, NI, N), jnp.float32),
        pltpu.SemaphoreType.DMA,
        pltpu.SemaphoreType.DMA,
    ],
)
def _sc_gather(p_hbm, q_hbm, src_hbm, dst_hbm, out_hbm,
               src_v, dst_v, p_v, q_v, ob0, ob1, sem0, sem1):
    wid = lax.axis_index("s") * NC + lax.axis_index("c")
    npw = RPW // 2 * N
    pltpu.async_copy(src_hbm, src_v, sem0)
    pltpu.async_copy(dst_hbm, dst_v, sem1)
    pltpu.async_copy(p_hbm.at[pl.ds(wid * npw, npw)], p_v, sem0)
    pltpu.async_copy(q_hbm.at[pl.ds(wid * npw, npw)], q_v, sem1)
    pltpu.make_async_copy(src_hbm, src_v, sem0).wait()
    pltpu.make_async_copy(dst_hbm, dst_v, sem1).wait()
    pltpu.make_async_copy(p_hbm.at[pl.ds(wid * npw, npw)], p_v, sem0).wait()
    pltpu.make_async_copy(q_hbm.at[pl.ds(wid * npw, npw)], q_v, sem1).wait()

    gpc = CHUNK // L

    def fill(ob, ci):
        @plsc.parallel_loop(0, gpc, 1, unroll=4)
        def _(it):
            si = it >> 3
            v2 = it & 7
            off = ci * CHUNK + it * L
            sv = src_v[pl.ds(off, L)]
            dv = dst_v[pl.ds(off, L)]
            for k in range(RPW // 2):
                pw = plsc.load_gather(p_v, [sv + (k * N)])
                qw = plsc.load_gather(q_v, [dv + (k * N)])
                pa, pb = plsc.unpack(
                    plsc.bitcast(pw, jnp.bfloat16),
                    format=plsc.PackFormat.INTERLEAVED,
                    preferred_element_type=jnp.float32)
                qa, qb = plsc.unpack(
                    plsc.bitcast(qw, jnp.bfloat16),
                    format=plsc.PackFormat.INTERLEAVED,
                    preferred_element_type=jnp.float32)
                s0 = pa + qa
                s1 = pb + qb
                ob[k, si, pl.ds(v2 * L, L)] = jnp.maximum(s0, 0.01 * s0)
                ob[k + RPW // 2, si, pl.ds(v2 * L, L)] = jnp.maximum(
                    s1, 0.01 * s1)

    def osl(ci):
        return out_hbm.at[wid, :, pl.ds(ci * NI, NI), :]

    def k_body(k, _):
        ci0, ci1 = 2 * k, 2 * k + 1

        @pl.when(k > 0)
        def _():
            pltpu.make_async_copy(ob0, osl(ci0), sem0).wait()

        fill(ob0, ci0)
        pltpu.async_copy(ob0, osl(ci0), sem0)

        @pl.when(k > 0)
        def _():
            pltpu.make_async_copy(ob1, osl(ci1), sem1).wait()

        fill(ob1, ci1)
        pltpu.async_copy(ob1, osl(ci1), sem1)
        return 0

    lax.fori_loop(0, NCHUNK // 2, k_body, 0)
    pltpu.make_async_copy(ob0, osl(0), sem0).wait()
    pltpu.make_async_copy(ob1, osl(1), sem1).wait()


def kernel(x, y, src, dst, tcm_w, tcm_b, n2e_w, n2e_b,
           e2s_w, e2s_b, e2d_w, e2d_b, n2e2_w, n2e2_b):
    xs = x.reshape(R, T, N)
    y_last = y[:, :, T - 1, :].reshape(R, N)
    wt_r = jnp.tile(tcm_w.reshape(1, C, T - 1), (BS, 1, 1)).reshape(R, T - 1)

    def per_row(v):
        return jnp.tile(v.reshape(1, C), (BS, 1)).reshape(R)

    params = jnp.stack([
        per_row(tcm_b),
        per_row(n2e_w[:, 0, 0, 0]), per_row(n2e_w[:, 0, 0, 1]),
        per_row(n2e_b),
        per_row(e2s_w[:, 0, 0, 0]), per_row(e2s_b),
        per_row(e2d_w[:, 0, 0, 0]), per_row(e2d_b),
        per_row(n2e2_w[:, 0, 0, 0]), per_row(n2e2_w[:, 0, 0, 1]),
        per_row(n2e2_w[:, 0, 0, 2]), per_row(n2e2_b),
    ], axis=1)

    partials = _sc_hist(src, dst)
    Pp, Qp = _tc_dense(xs, y_last, wt_r, params,
                       partials.reshape(NW, N, N))
    return _sc_gather(Pp.reshape(R // 2 * N), Qp.reshape(R // 2 * N),
                      src, dst)
